# baseline (speedup 1.0000x reference)
import functools

import jax
import jax.numpy as jnp
from jax import lax
from jax.experimental import pallas as pl
from jax.experimental.pallas import tpu as pltpu
from jax.experimental.pallas import tpu_sc as plsc

VOCAB = 65
EMBED = 128
BATCH = 16384
CTX = 10
JPAD = 80
RPAD = 128
KOFF = 33
NPAIR = 33
TROWS = 40

JB = 9

NC = 2
NS = 16
NW = NC * NS
ROWS_PER = BATCH // NW
GROUPS = ROWS_PER // 16


def _fuse_table_body(emb_ref, w_ref, bcol_ref, out_ref):
    mm = lax.dot_general(
        w_ref[...], emb_ref[...],
        (((1,), (1,)), ((), ())),
        preferred_element_type=jnp.float32,
    )
    bcol = bcol_ref[...].reshape(VOCAB, 1)
    tt = jnp.pad((mm + bcol) * (1.0 / CTX),
                 ((0, JPAD - VOCAB), (0, RPAD - VOCAB)))
    lo = lax.convert_element_type(tt[:TROWS], jnp.bfloat16)
    hi = lax.convert_element_type(tt[KOFF:KOFF + TROWS], jnp.bfloat16)
    lo32 = lax.convert_element_type(
        lax.bitcast_convert_type(lo, jnp.uint16), jnp.uint32)
    hi32 = lax.convert_element_type(
        lax.bitcast_convert_type(hi, jnp.uint16), jnp.uint32)
    packed = lo32 | (hi32 << 16)
    out_ref[...] = lax.bitcast_convert_type(packed, jnp.int32)


def _build_table_packed(embedding, W, b):
    return pl.pallas_call(
        _fuse_table_body,
        out_shape=jax.ShapeDtypeStruct((TROWS, RPAD), jnp.int32),
    )(embedding, W, b)


_sc_mesh = plsc.VectorSubcoreMesh(core_axis_name="c", subcore_axis_name="s")

_HIMASK = -65536


@functools.partial(
    pl.kernel,
    mesh=_sc_mesh,
    compiler_params=pltpu.CompilerParams(needs_layout_passes=False),
    out_type=jax.ShapeDtypeStruct((JB * BATCH * 8,), jnp.float32),
    scratch_types=[
        pltpu.VMEM((CTX * ROWS_PER,), jnp.int32),
        pltpu.VMEM((TROWS * RPAD,), jnp.int32),
        pltpu.VMEM((JB * 4 * 8 * 128,), jnp.float32),
        pltpu.SemaphoreType.DMA,
        pltpu.SemaphoreType.DMA,
    ],
)
def _sc_pool(ctx_hbm, tab_hbm, out_hbm, ctx_v, tab_v, out_v, in_sem, out_sem):
    wid = lax.axis_index("s") * NC + lax.axis_index("c")
    base = wid * ROWS_PER

    in_copies = [
        pltpu.make_async_copy(
            ctx_hbm.at[pl.ds(c * BATCH + base, ROWS_PER)],
            ctx_v.at[pl.ds(c * ROWS_PER, ROWS_PER)],
            in_sem,
        )
        for c in range(CTX)
    ]
    in_copies.append(pltpu.make_async_copy(tab_hbm, tab_v, in_sem))
    for cp in in_copies:
        cp.start()
    for cp in in_copies:
        cp.wait()

    def unpack_pair_sum(word):
        lo = plsc.bitcast(word << 16, jnp.float32)
        hi = plsc.bitcast(word & _HIMASK, jnp.float32)
        return lo, hi

    @plsc.parallel_loop(0, GROUPS, unroll=2)
    def per_group(g):
        rb = g * 16
        srow = (g // 8) * 1024 + (g % 8) * 16
        idxs = [ctx_v[pl.ds(c * ROWS_PER + rb, 16)] for c in range(CTX)]

        def gathers(jp):
            row = tab_v.at[pl.ds(jp * RPAD, RPAD)]
            return [plsc.load_gather(row, [idxs[c]]) for c in range(CTX)]

        pipe = [gathers(0), gathers(1)]
        for jp in range(NPAIR):
            gs = pipe[0]
            pipe = [pipe[1], gathers(jp + 2) if jp + 2 < NPAIR else None]
            sums = []
            for k in range(CTX // 2):
                a = plsc.bitcast(gs[2 * k], jnp.bfloat16)
                b16 = plsc.bitcast(gs[2 * k + 1], jnp.bfloat16)
                sums.append(plsc.bitcast(a + b16, jnp.int32))
            los, his = zip(*(unpack_pair_sum(s) for s in sums))
            acc_lo = (los[0] + los[1]) + (los[2] + los[3]) + los[4]
            acc_hi = (his[0] + his[1]) + (his[2] + his[3]) + his[4]
            jlo, jhi = jp, jp + KOFF
            out_v[pl.ds((jlo // 8) * 4096 + (jlo % 8) * 128 + srow, 16)] = acc_lo
            if jhi < VOCAB:
                out_v[pl.ds((jhi // 8) * 4096 + (jhi % 8) * 128 + srow, 16)] = acc_hi

    out_copies = [
        pltpu.make_async_copy(
            out_v.at[pl.ds(jb * 4096, 4096)],
            out_hbm.at[pl.ds(jb * BATCH * 8 + base * 8, 4096)],
            out_sem,
        )
        for jb in range(JB)
    ]
    for cp in out_copies:
        cp.start()
    for cp in out_copies:
        cp.wait()


def kernel(context, embedding, W, b):
    tab_packed = _build_table_packed(embedding, W, b)
    ctx_t_flat = context.T.reshape(CTX * BATCH)
    out_flat = _sc_pool(ctx_t_flat, tab_packed.reshape(TROWS * RPAD))
    out4 = out_flat.reshape(JB, BATCH // 128, 8, 128)
    return out4.transpose(1, 3, 0, 2).reshape(BATCH, JB * 8)[:, :VOCAB]


# speedup vs baseline: 1.0156x; 1.0156x over previous
import functools

import jax
import jax.numpy as jnp
from jax import lax
from jax.experimental import pallas as pl
from jax.experimental.pallas import tpu as pltpu
from jax.experimental.pallas import tpu_sc as plsc

VOCAB = 65
EMBED = 128
BATCH = 16384
CTX = 10
JPAD = 80
RPAD = 128
KOFF = 33
NPAIR = 33
TROWS = 40

JB = 9

NC = 2
NS = 16
NW = NC * NS
ROWS_PER = BATCH // NW
GROUPS = ROWS_PER // 16


def _fuse_table_body(emb_ref, w_ref, bcol_ref, out_ref):
    mm = lax.dot_general(
        w_ref[...], emb_ref[...],
        (((1,), (1,)), ((), ())),
        preferred_element_type=jnp.float32,
    )
    bcol = bcol_ref[...].reshape(VOCAB, 1)
    tt = jnp.pad((mm + bcol) * (1.0 / CTX),
                 ((0, JPAD - VOCAB), (0, RPAD - VOCAB)))
    lo = lax.convert_element_type(tt[:TROWS], jnp.bfloat16)
    hi = lax.convert_element_type(tt[KOFF:KOFF + TROWS], jnp.bfloat16)
    lo32 = lax.convert_element_type(
        lax.bitcast_convert_type(lo, jnp.uint16), jnp.uint32)
    hi32 = lax.convert_element_type(
        lax.bitcast_convert_type(hi, jnp.uint16), jnp.uint32)
    packed = lo32 | (hi32 << 16)
    out_ref[...] = lax.bitcast_convert_type(packed, jnp.int32)


def _build_table_packed(embedding, W, b):
    return pl.pallas_call(
        _fuse_table_body,
        out_shape=jax.ShapeDtypeStruct((TROWS, RPAD), jnp.int32),
    )(embedding, W, b)


_sc_mesh = plsc.VectorSubcoreMesh(core_axis_name="c", subcore_axis_name="s")

_HIMASK = -65536


@functools.partial(
    pl.kernel,
    mesh=_sc_mesh,
    compiler_params=pltpu.CompilerParams(needs_layout_passes=False),
    out_type=jax.ShapeDtypeStruct((JB * BATCH * 8,), jnp.float32),
    scratch_types=[
        pltpu.VMEM((CTX * ROWS_PER,), jnp.int32),
        pltpu.VMEM((TROWS * RPAD,), jnp.int32),
        pltpu.VMEM((JB * 4 * 8 * 128,), jnp.float32),
        pltpu.SemaphoreType.DMA,
        pltpu.SemaphoreType.DMA,
    ],
)
def _sc_pool(ctx_hbm, tab_hbm, out_hbm, ctx_v, tab_v, out_v, in_sem, out_sem):
    wid = lax.axis_index("s") * NC + lax.axis_index("c")
    base = wid * ROWS_PER

    in_copies = [
        pltpu.make_async_copy(
            ctx_hbm.at[pl.ds(c * BATCH + base, ROWS_PER)],
            ctx_v.at[pl.ds(c * ROWS_PER, ROWS_PER)],
            in_sem,
        )
        for c in range(CTX)
    ]
    in_copies.append(pltpu.make_async_copy(tab_hbm, tab_v, in_sem))
    for cp in in_copies:
        cp.start()
    for cp in in_copies:
        cp.wait()

    def unpack_pair_sum(word):
        lo = plsc.bitcast(word << 16, jnp.float32)
        hi = plsc.bitcast(word & _HIMASK, jnp.float32)
        return lo, hi

    @plsc.parallel_loop(0, GROUPS, unroll=1)
    def per_group(g):
        rb = g * 16
        srow = (g // 8) * 1024 + (g % 8) * 16
        idxs = [ctx_v[pl.ds(c * ROWS_PER + rb, 16)] for c in range(CTX)]

        def gathers(jp):
            row = tab_v.at[pl.ds(jp * RPAD, RPAD)]
            return [plsc.load_gather(row, [idxs[c]]) for c in range(CTX)]

        pipe = [gathers(0), gathers(1)]
        for jp in range(NPAIR):
            gs = pipe[0]
            pipe = [pipe[1], gathers(jp + 2) if jp + 2 < NPAIR else None]
            sums = []
            for k in range(CTX // 2):
                a = plsc.bitcast(gs[2 * k], jnp.bfloat16)
                b16 = plsc.bitcast(gs[2 * k + 1], jnp.bfloat16)
                sums.append(plsc.bitcast(a + b16, jnp.int32))
            los, his = zip(*(unpack_pair_sum(s) for s in sums))
            acc_lo = (los[0] + los[1]) + (los[2] + los[3]) + los[4]
            acc_hi = (his[0] + his[1]) + (his[2] + his[3]) + his[4]
            jlo, jhi = jp, jp + KOFF
            out_v[pl.ds((jlo // 8) * 4096 + (jlo % 8) * 128 + srow, 16)] = acc_lo
            if jhi < VOCAB:
                out_v[pl.ds((jhi // 8) * 4096 + (jhi % 8) * 128 + srow, 16)] = acc_hi

    out_copies = [
        pltpu.make_async_copy(
            out_v.at[pl.ds(jb * 4096, 4096)],
            out_hbm.at[pl.ds(jb * BATCH * 8 + base * 8, 4096)],
            out_sem,
        )
        for jb in range(JB)
    ]
    for cp in out_copies:
        cp.start()
    for cp in out_copies:
        cp.wait()


def kernel(context, embedding, W, b):
    tab_packed = _build_table_packed(embedding, W, b)
    ctx_t_flat = context.T.reshape(CTX * BATCH)
    out_flat = _sc_pool(ctx_t_flat, tab_packed.reshape(TROWS * RPAD))
    out4 = out_flat.reshape(JB, BATCH // 128, 8, 128)
    return out4.transpose(1, 3, 0, 2).reshape(BATCH, JB * 8)[:, :VOCAB]


# speedup vs baseline: 1.0164x; 1.0008x over previous
import functools

import jax
import jax.numpy as jnp
from jax import lax
from jax.experimental import pallas as pl
from jax.experimental.pallas import tpu as pltpu
from jax.experimental.pallas import tpu_sc as plsc

VOCAB = 65
EMBED = 128
BATCH = 16384
CTX = 10
JPAD = 80
RPAD = 128
KOFF = 33
NPAIR = 33
TROWS = 40

JB = 9

NC = 2
NS = 16
NW = NC * NS
ROWS_PER = BATCH // NW
GROUPS = ROWS_PER // 16


def _fuse_table_body(emb_ref, w_ref, bcol_ref, out_ref):
    mm = lax.dot_general(
        w_ref[...], emb_ref[...],
        (((1,), (1,)), ((), ())),
        preferred_element_type=jnp.float32,
    )
    bcol = bcol_ref[...].reshape(VOCAB, 1)
    tt = jnp.pad((mm + bcol) * (1.0 / CTX),
                 ((0, JPAD - VOCAB), (0, RPAD - VOCAB)))
    lo = lax.convert_element_type(tt[:TROWS], jnp.bfloat16)
    hi = lax.convert_element_type(tt[KOFF:KOFF + TROWS], jnp.bfloat16)
    lo32 = lax.convert_element_type(
        lax.bitcast_convert_type(lo, jnp.uint16), jnp.uint32)
    hi32 = lax.convert_element_type(
        lax.bitcast_convert_type(hi, jnp.uint16), jnp.uint32)
    packed = lo32 | (hi32 << 16)
    out_ref[...] = lax.bitcast_convert_type(packed, jnp.int32)


def _build_table_packed(embedding, W, b):
    return pl.pallas_call(
        _fuse_table_body,
        out_shape=jax.ShapeDtypeStruct((TROWS, RPAD), jnp.int32),
    )(embedding, W, b)


_sc_mesh = plsc.VectorSubcoreMesh(core_axis_name="c", subcore_axis_name="s")

_HIMASK = -65536


@functools.partial(
    pl.kernel,
    mesh=_sc_mesh,
    compiler_params=pltpu.CompilerParams(needs_layout_passes=False),
    out_type=jax.ShapeDtypeStruct((JB * BATCH * 8,), jnp.float32),
    scratch_types=[
        pltpu.VMEM((CTX * ROWS_PER,), jnp.int32),
        pltpu.VMEM((TROWS * RPAD,), jnp.int32),
        pltpu.VMEM((JB * 4 * 8 * 128,), jnp.float32),
        pltpu.SemaphoreType.DMA,
        pltpu.SemaphoreType.DMA,
    ],
)
def _sc_pool(ctx_hbm, tab_hbm, out_hbm, ctx_v, tab_v, out_v, in_sem, out_sem):
    wid = lax.axis_index("s") * NC + lax.axis_index("c")
    base = wid * ROWS_PER

    in_copies = [
        pltpu.make_async_copy(
            ctx_hbm.at[pl.ds(c * BATCH + base, ROWS_PER)],
            ctx_v.at[pl.ds(c * ROWS_PER, ROWS_PER)],
            in_sem,
        )
        for c in range(CTX)
    ]
    in_copies.append(pltpu.make_async_copy(tab_hbm, tab_v, in_sem))
    for cp in in_copies:
        cp.start()
    for cp in in_copies:
        cp.wait()

    def unpack_pair_sum(word):
        lo = plsc.bitcast(word << 16, jnp.float32)
        hi = plsc.bitcast(word & _HIMASK, jnp.float32)
        return lo, hi

    @plsc.parallel_loop(0, GROUPS, unroll=1)
    def per_group(g):
        rb = g * 16
        srow = (g // 8) * 1024 + (g % 8) * 16
        idxs = [ctx_v[pl.ds(c * ROWS_PER + rb, 16)] for c in range(CTX)]

        def gathers(jp):
            row = tab_v.at[pl.ds(jp * RPAD, RPAD)]
            return [plsc.load_gather(row, [idxs[c]]) for c in range(CTX)]

        pipe = [gathers(0), gathers(1), gathers(2)]
        for jp in range(NPAIR):
            gs = pipe[0]
            pipe = [pipe[1], pipe[2],
                    gathers(jp + 3) if jp + 3 < NPAIR else None]
            sums = []
            for k in range(CTX // 2):
                a = plsc.bitcast(gs[2 * k], jnp.bfloat16)
                b16 = plsc.bitcast(gs[2 * k + 1], jnp.bfloat16)
                sums.append(plsc.bitcast(a + b16, jnp.int32))
            los, his = zip(*(unpack_pair_sum(s) for s in sums))
            acc_lo = (los[0] + los[1]) + (los[2] + los[3]) + los[4]
            acc_hi = (his[0] + his[1]) + (his[2] + his[3]) + his[4]
            jlo, jhi = jp, jp + KOFF
            out_v[pl.ds((jlo // 8) * 4096 + (jlo % 8) * 128 + srow, 16)] = acc_lo
            if jhi < VOCAB:
                out_v[pl.ds((jhi // 8) * 4096 + (jhi % 8) * 128 + srow, 16)] = acc_hi

    out_copies = [
        pltpu.make_async_copy(
            out_v.at[pl.ds(jb * 4096, 4096)],
            out_hbm.at[pl.ds(jb * BATCH * 8 + base * 8, 4096)],
            out_sem,
        )
        for jb in range(JB)
    ]
    for cp in out_copies:
        cp.start()
    for cp in out_copies:
        cp.wait()


def kernel(context, embedding, W, b):
    tab_packed = _build_table_packed(embedding, W, b)
    ctx_t_flat = context.T.reshape(CTX * BATCH)
    out_flat = _sc_pool(ctx_t_flat, tab_packed.reshape(TROWS * RPAD))
    out4 = out_flat.reshape(JB, BATCH // 128, 8, 128)
    return out4.transpose(1, 3, 0, 2).reshape(BATCH, JB * 8)[:, :VOCAB]
